# baseline (speedup 1.0000x reference)
import functools

import jax
import jax.numpy as jnp
from jax import lax
from jax.experimental import pallas as pl

_ANCHOR_W = (10.0, 16.0, 33.0)
_ANCHOR_H = (13.0, 30.0, 23.0)
_IMG_DIM = 608.0


def _body(x0_ref, x1_ref, x2_ref, o_ref, *, bs, in_h, stride):
    cb = pl.program_id(0)
    hw = in_h * in_h
    refs = (x0_ref, x1_ref, x2_ref)

    n = lax.broadcasted_iota(jnp.int32, (1, hw), 1)
    gx = (n % in_h).astype(jnp.float32)
    gy = (n // in_h).astype(jnp.float32)

    for a in range(3):
        sl = pl.ds(a * hw, hw)
        for i in range(5):
            v = refs[a][:, i].reshape(bs, hw)
            o_ref[i, :, sl] = jax.nn.sigmoid(v)

        @pl.when(cb == 0)
        def _(a=a, sl=sl):
            v0 = refs[a][:, 0].reshape(bs, hw)
            v1 = refs[a][:, 1].reshape(bs, hw)
            v2 = refs[a][:, 2].reshape(bs, hw)
            v3 = refs[a][:, 3].reshape(bs, hw)
            o_ref[0, :, sl] = (jax.nn.sigmoid(v0) + gx) * stride
            o_ref[1, :, sl] = (jax.nn.sigmoid(v1) + gy) * stride
            o_ref[2, :, sl] = jnp.exp(v2) * _ANCHOR_W[a]
            o_ref[3, :, sl] = jnp.exp(v3) * _ANCHOR_H[a]


def kernel(x):
    bs, ch, in_h, _ = x.shape
    na = 3
    attrs = ch // na
    hw = in_h * in_h
    stride = _IMG_DIM / in_h
    cgrp = 5
    ngrp = attrs // cgrp

    body = functools.partial(_body, bs=bs, in_h=in_h, stride=stride)

    x4 = x.reshape(bs, ch, 8, hw // 8)

    def in_spec(a):
        return pl.BlockSpec(
            (bs, cgrp, 8, hw // 8), lambda cb, a=a: (0, a * ngrp + cb, 0, 0)
        )

    out = pl.pallas_call(
        body,
        grid=(ngrp,),
        in_specs=[in_spec(0), in_spec(1), in_spec(2)],
        out_specs=pl.BlockSpec((cgrp, bs, na * hw), lambda cb: (cb, 0, 0)),
        out_shape=jax.ShapeDtypeStruct((attrs, bs, na * hw), jnp.float32),
    )(x4, x4, x4)
    return out.transpose(1, 2, 0)


# speedup vs baseline: 1.0883x; 1.0883x over previous
import functools

import jax
import jax.numpy as jnp
from jax import lax
from jax.experimental import pallas as pl

_ANCHOR_W = (10.0, 16.0, 33.0)
_ANCHOR_H = (13.0, 30.0, 23.0)
_IMG_DIM = 608.0


def _body(x0_ref, x1_ref, x2_ref, o_ref, *, bs, in_h, stride):
    cb = pl.program_id(0)
    hw = in_h * in_h
    refs = (x0_ref, x1_ref, x2_ref)

    n = lax.broadcasted_iota(jnp.int32, (1, hw), 1)
    gx = (n % in_h).astype(jnp.float32)
    gy = (n // in_h).astype(jnp.float32)

    for a in range(3):
        sl = pl.ds(a * hw, hw)
        for i in range(5):
            v = refs[a][:, i].reshape(bs, hw)
            o_ref[i, :, sl] = jax.nn.sigmoid(v)

        @pl.when(cb == 0)
        def _(a=a, sl=sl):
            v0 = refs[a][:, 0].reshape(bs, hw)
            v1 = refs[a][:, 1].reshape(bs, hw)
            v2 = refs[a][:, 2].reshape(bs, hw)
            v3 = refs[a][:, 3].reshape(bs, hw)
            o_ref[0, :, sl] = (jax.nn.sigmoid(v0) + gx) * stride
            o_ref[1, :, sl] = (jax.nn.sigmoid(v1) + gy) * stride
            o_ref[2, :, sl] = jnp.exp(v2) * _ANCHOR_W[a]
            o_ref[3, :, sl] = jnp.exp(v3) * _ANCHOR_H[a]


def kernel(x):
    bs, ch, in_h, _ = x.shape
    na = 3
    attrs = ch // na
    hw = in_h * in_h
    stride = _IMG_DIM / in_h
    cgrp = 5
    ngrp = attrs // cgrp

    body = functools.partial(_body, bs=bs, in_h=in_h, stride=stride)

    def in_spec(a):
        return pl.BlockSpec(
            (bs, cgrp, in_h, in_h), lambda cb, a=a: (0, a * ngrp + cb, 0, 0)
        )

    out = pl.pallas_call(
        body,
        grid=(ngrp,),
        in_specs=[in_spec(0), in_spec(1), in_spec(2)],
        out_specs=pl.BlockSpec((cgrp, bs, na * hw), lambda cb: (cb, 0, 0)),
        out_shape=jax.ShapeDtypeStruct((attrs, bs, na * hw), jnp.float32),
    )(x, x, x)
    return out.transpose(1, 2, 0)
